# baseline (speedup 1.0000x reference)
import functools

import jax
import jax.numpy as jnp
from jax import lax
from jax.experimental import pallas as pl
from jax.experimental.pallas import tpu as pltpu
from jax.experimental.pallas import tpu_sc as plsc

_NC = 2
_NS = 16
_NW = _NC * _NS

_TV = 2048
_NBUF = 2


def _sc_gather_packed(packed, idx):
    B = idx.shape[0]
    P = packed.shape[1]
    b_per_w = B // _NW
    mesh = plsc.VectorSubcoreMesh(
        core_axis_name="c", subcore_axis_name="s",
        num_cores=_NC, num_subcores=_NS)

    @functools.partial(
        pl.kernel,
        out_type=jax.ShapeDtypeStruct((B, P), jnp.float32),
        mesh=mesh,
        scratch_types=[
            pltpu.VMEM((b_per_w,), jnp.int32),
            pltpu.VMEM((b_per_w,), jnp.int32),
            pltpu.VMEM((b_per_w, P), jnp.float32),
            pltpu.SemaphoreType.DMA,
        ],
    )
    def gather_kernel(packed_hbm, idx_hbm, out_hbm, idx_v, idx2_v, rows_v, sem):
        wid = lax.axis_index("s") * _NC + lax.axis_index("c")
        base = wid * b_per_w
        pltpu.sync_copy(idx_hbm.at[pl.ds(base, b_per_w)], idx_v)
        for k in range(b_per_w // 16):
            sl = pl.ds(k * 16, 16)
            idx2_v[sl] = lax.shift_right_logical(idx_v[sl], 1)
        pltpu.async_copy(packed_hbm.at[idx2_v], rows_v, sem).wait()
        pltpu.sync_copy(rows_v, out_hbm.at[pl.ds(base, b_per_w)])

    return gather_kernel(packed, idx)


_TB = 32


def _make_mm_body(B, V):
    nb = B // _TB

    def _mm_body(e_ref, p_ref, w_ref, b_ref, o_hbm, bufs, sems):
        i = pl.program_id(0)
        slot = lax.rem(i, _NBUF)

        @pl.when(i >= _NBUF)
        def _():
            pltpu.make_async_copy(
                bufs.at[slot],
                o_hbm.at[pl.ds((i - _NBUF) * _TB, _TB), :],
                sems.at[slot],
            ).wait()

        ep = e_ref[...]
        D = ep.shape[1] // 2
        odd = p_ref[...] != 0
        e = jnp.where(odd, ep[:, D:], ep[:, :D])
        bufs[slot] = lax.dot_general(
            e, w_ref[...],
            dimension_numbers=(((1,), (0,)), ((), ())),
            preferred_element_type=jnp.float32,
        ) + b_ref[0]

        pltpu.make_async_copy(
            bufs.at[slot],
            o_hbm.at[pl.ds(i * _TB, _TB), :],
            sems.at[slot],
        ).start()

        @pl.when(i == nb - 1)
        def _():
            for k in range(_NBUF):
                pltpu.make_async_copy(
                    bufs.at[k],
                    o_hbm.at[pl.ds(k * _TB, _TB), :],
                    sems.at[k],
                ).wait()

    return _mm_body, nb


def _tc_project(epacked, parity, W, b):
    B, P = epacked.shape
    V, D = W.shape
    Wt = W.T
    body, nb = _make_mm_body(B, V)
    return pl.pallas_call(
        body,
        grid=(nb,),
        in_specs=[
            pl.BlockSpec((_TB, P), lambda i: (i, 0)),
            pl.BlockSpec((_TB, 1), lambda i: (i, 0)),
            pl.BlockSpec((D, V), lambda i: (0, 0)),
            pl.BlockSpec((1, V), lambda i: (0, 0)),
        ],
        out_specs=pl.BlockSpec(memory_space=pl.ANY),
        out_shape=jax.ShapeDtypeStruct((B, V), jnp.float32),
        scratch_shapes=[
            pltpu.VMEM((_NBUF, _TB, V), jnp.float32),
            pltpu.SemaphoreType.DMA((_NBUF,)),
        ],
        compiler_params=pltpu.CompilerParams(
            vmem_limit_bytes=110 * 1024 * 1024),
    )(epacked, parity, Wt, b.reshape(1, V))


def kernel(x, table, W, b):
    xi = x.astype(jnp.int32)
    V, D = table.shape
    packed = table.reshape(V // 2, 2 * D)
    epacked = _sc_gather_packed(packed, xi)
    parity = (xi & 1).astype(jnp.int32).reshape(-1, 1)
    return _tc_project(epacked, parity, W, b)


# speedup vs baseline: 1.7662x; 1.7662x over previous
import functools

import jax
import jax.numpy as jnp
from jax import lax
from jax.experimental import pallas as pl
from jax.experimental.pallas import tpu as pltpu
from jax.experimental.pallas import tpu_sc as plsc

_NC = 2
_NS = 16
_NW = _NC * _NS

_TV = 2000
_NBUF = 4


def _sc_gather_packed(packed, idx):
    B = idx.shape[0]
    P = packed.shape[1]
    b_per_w = B // _NW
    mesh = plsc.VectorSubcoreMesh(
        core_axis_name="c", subcore_axis_name="s",
        num_cores=_NC, num_subcores=_NS)

    @functools.partial(
        pl.kernel,
        out_type=jax.ShapeDtypeStruct((B, P), jnp.float32),
        mesh=mesh,
        scratch_types=[
            pltpu.VMEM((b_per_w,), jnp.int32),
            pltpu.VMEM((b_per_w,), jnp.int32),
            pltpu.VMEM((b_per_w, P), jnp.float32),
            pltpu.SemaphoreType.DMA,
        ],
    )
    def gather_kernel(packed_hbm, idx_hbm, out_hbm, idx_v, idx2_v, rows_v, sem):
        wid = lax.axis_index("s") * _NC + lax.axis_index("c")
        base = wid * b_per_w
        pltpu.sync_copy(idx_hbm.at[pl.ds(base, b_per_w)], idx_v)
        for k in range(b_per_w // 16):
            sl = pl.ds(k * 16, 16)
            idx2_v[sl] = lax.shift_right_logical(idx_v[sl], 1)
        pltpu.async_copy(packed_hbm.at[idx2_v], rows_v, sem).wait()
        pltpu.sync_copy(rows_v, out_hbm.at[pl.ds(base, b_per_w)])

    return gather_kernel(packed, idx)


def _make_mm_body(B, V, nv):
    def _mm_body(w_ref, e_ref, p_ref, b_ref, o_hbm, bufs, sems):
        j = pl.program_id(0)
        slot = lax.rem(j, _NBUF)

        @pl.when(j >= _NBUF)
        def _():
            pltpu.make_async_copy(
                bufs.at[slot],
                o_hbm.at[pl.ds((j - _NBUF) * _TV, _TV), :],
                sems.at[slot],
            ).wait()

        ep = e_ref[...]
        D = ep.shape[1] // 2
        odd = p_ref[...] != 0
        e = jnp.where(odd, ep[:, D:], ep[:, :D])
        bufs[slot] = lax.dot_general(
            w_ref[...], e,
            dimension_numbers=(((1,), (1,)), ((), ())),
            preferred_element_type=jnp.float32,
        ) + b_ref[...]

        pltpu.make_async_copy(
            bufs.at[slot],
            o_hbm.at[pl.ds(j * _TV, _TV), :],
            sems.at[slot],
        ).start()

        @pl.when(j == nv - 1)
        def _():
            for k in range(_NBUF):
                pltpu.make_async_copy(
                    bufs.at[k],
                    o_hbm.at[pl.ds(k * _TV, _TV), :],
                    sems.at[k],
                ).wait()

    return _mm_body


def _tc_project(epacked, parity, W, b):
    B, P = epacked.shape
    V, D = W.shape
    nv = V // _TV
    body = _make_mm_body(B, V, nv)
    outT = pl.pallas_call(
        body,
        grid=(nv,),
        in_specs=[
            pl.BlockSpec((_TV, D), lambda j: (j, 0)),
            pl.BlockSpec((B, P), lambda j: (0, 0)),
            pl.BlockSpec((B, 1), lambda j: (0, 0)),
            pl.BlockSpec((_TV, 1), lambda j: (j, 0)),
        ],
        out_specs=pl.BlockSpec(memory_space=pl.ANY),
        out_shape=jax.ShapeDtypeStruct((V, B), jnp.float32),
        scratch_shapes=[
            pltpu.VMEM((_NBUF, _TV, B), jnp.float32),
            pltpu.SemaphoreType.DMA((_NBUF,)),
        ],
        compiler_params=pltpu.CompilerParams(
            vmem_limit_bytes=110 * 1024 * 1024),
    )(W, epacked, parity, b.reshape(V, 1))
    return outT.T


def kernel(x, table, W, b):
    xi = x.astype(jnp.int32)
    V, D = table.shape
    packed = table.reshape(V // 2, 2 * D)
    epacked = _sc_gather_packed(packed, xi)
    parity = (xi & 1).astype(jnp.int32).reshape(-1, 1)
    return _tc_project(epacked, parity, W, b)
